# baseline (speedup 1.0000x reference)
import functools

import jax
import jax.numpy as jnp
from jax import lax
from jax.experimental import pallas as pl
from jax.experimental.pallas import tpu as pltpu
from jax.experimental.pallas import tpu_sc as plsc

N_NODE = 10000
N_EDGE = 320000
EMB = 128
N_SESS = 1142

NC = 2
NS = 16
NW = NC * NS
CHUNK = 96
NCHK = 106
EPT = CHUNK * NCHK
EPAD = EPT * NW
WTILES = 10
RPT = N_NODE // WTILES
ZR = 200


def _edge_pass_body(rows_hbm, cols_hbm, vals_hbm, table_hbm, p0_hbm, p1_hbm,
                    idx_c0, idx_c1, idx_r0, idx_r1, val0, val1,
                    rowbuf0, rowbuf1, zbuf, acc,
                    semi0, semi1, semg0, semg1):
    c = lax.axis_index("c")
    s = lax.axis_index("s")
    wid = s * NC + c

    zv = jnp.zeros((16,), jnp.float32)

    def zrow(i, carry):
        for j in range(8):
            zbuf[i, pl.ds(j * 16, 16)] = zv
        return carry

    lax.fori_loop(0, ZR, zrow, 0)

    @pl.when(s < WTILES)
    def _():
        for q in range(RPT // ZR):
            off = pl.multiple_of(s * RPT + q * ZR, 8)
            pltpu.sync_copy(zbuf, acc.at[pl.ds(off, ZR)])

    plsc.subcore_barrier()

    idx_c = (idx_c0, idx_c1)
    idx_r = (idx_r0, idx_r1)
    val = (val0, val1)
    rowbuf = (rowbuf0, rowbuf1)
    semi = (semi0, semi1)
    semg = (semg0, semg1)

    def ebase(q):
        return pl.multiple_of(wid * EPT + q * CHUNK, 8)

    def issue_idx(q, b):
        pltpu.async_copy(cols_hbm.at[pl.ds(ebase(q), CHUNK)], idx_c[b], semi[b])
        pltpu.async_copy(rows_hbm.at[pl.ds(ebase(q), CHUNK)], idx_r[b], semi[b])
        pltpu.async_copy(vals_hbm.at[pl.ds(ebase(q), CHUNK)], val[b], semi[b])

    def wait_idx(b):
        pltpu.make_async_copy(cols_hbm.at[pl.ds(0, CHUNK)], idx_c[b],
                              semi[b]).wait()
        pltpu.make_async_copy(rows_hbm.at[pl.ds(0, CHUNK)], idx_r[b],
                              semi[b]).wait()
        pltpu.make_async_copy(vals_hbm.at[pl.ds(0, CHUNK)], val[b],
                              semi[b]).wait()

    def issue_gather(b):
        pltpu.async_copy(table_hbm.at[idx_c[b]], rowbuf[b], semg[b])

    def wait_gather(b):
        pltpu.make_async_copy(table_hbm.at[pl.ds(0, CHUNK)], rowbuf[b],
                              semg[b]).wait()

    def process(b):
        @plsc.parallel_loop(0, CHUNK // 16, unroll=2)
        def _(g):
            vv = val[b][pl.ds(g * 16, 16)]
            for l in range(16):
                v = vv[l]
                e = g * 16 + l
                for j in range(8):
                    rowbuf[b][e, pl.ds(j * 16, 16)] = (
                        rowbuf[b][e, pl.ds(j * 16, 16)] * v)

        pltpu.sync_copy(rowbuf[b], acc.at[idx_r[b]], add=True)

    def slot_pair(k, carry):
        for b in range(2):
            t = k * 2 + b

            @pl.when(jnp.logical_and(t >= 1, t <= NCHK))
            def _():
                wait_idx(1 - b)
                issue_gather(1 - b)

            @pl.when(t >= 2)
            def _():
                wait_gather(b)
                process(b)

            @pl.when(t <= NCHK - 1)
            def _():
                issue_idx(t, b)
        return carry

    lax.fori_loop(0, NCHK // 2 + 1, slot_pair, 0)
    plsc.subcore_barrier()

    woff = pl.multiple_of(s * RPT, 8)

    @pl.when(jnp.logical_and(s < WTILES, c == 0))
    def _():
        pltpu.sync_copy(acc.at[pl.ds(woff, RPT)], p0_hbm.at[pl.ds(woff, RPT)])

    @pl.when(jnp.logical_and(s < WTILES, c == 1))
    def _():
        pltpu.sync_copy(acc.at[pl.ds(woff, RPT)], p1_hbm.at[pl.ds(woff, RPT)])


_edge_pass = functools.partial(
    pl.kernel,
    out_type=(jax.ShapeDtypeStruct((N_NODE, EMB), jnp.float32),
              jax.ShapeDtypeStruct((N_NODE, EMB), jnp.float32)),
    mesh=plsc.VectorSubcoreMesh(core_axis_name="c", subcore_axis_name="s"),
    scratch_types=[
        pltpu.VMEM((CHUNK,), jnp.int32),
        pltpu.VMEM((CHUNK,), jnp.int32),
        pltpu.VMEM((CHUNK,), jnp.int32),
        pltpu.VMEM((CHUNK,), jnp.int32),
        pltpu.VMEM((CHUNK,), jnp.float32),
        pltpu.VMEM((CHUNK,), jnp.float32),
        pltpu.VMEM((CHUNK, EMB), jnp.float32),
        pltpu.VMEM((CHUNK, EMB), jnp.float32),
        pltpu.VMEM((ZR, EMB), jnp.float32),
        pltpu.VMEM_SHARED((N_NODE, EMB), jnp.float32),
        pltpu.SemaphoreType.DMA,
        pltpu.SemaphoreType.DMA,
        pltpu.SemaphoreType.DMA,
        pltpu.SemaphoreType.DMA,
    ],
)(_edge_pass_body)


def _line_body(d_ref, a_ref, e1_ref, out_ref):
    da = jnp.dot(d_ref[...], a_ref[...], preferred_element_type=jnp.float32)
    y1 = jnp.dot(da, e1_ref[...], preferred_element_type=jnp.float32)
    y2 = jnp.dot(da, y1, preferred_element_type=jnp.float32)
    out_ref[...] = e1_ref[...] + y1 + y2


_line = pl.pallas_call(
    _line_body,
    out_shape=jax.ShapeDtypeStruct((N_SESS, EMB), jnp.float32),
    in_specs=[pl.BlockSpec(memory_space=pltpu.VMEM)] * 3,
    out_specs=pl.BlockSpec(memory_space=pltpu.VMEM),
)

BM = 1000


def _add_body(a_ref, b_ref, o_ref):
    o_ref[...] = a_ref[...] + b_ref[...]


_combine = pl.pallas_call(
    _add_body,
    grid=(N_NODE // BM,),
    in_specs=[pl.BlockSpec((BM, EMB), lambda i: (i, 0))] * 2,
    out_specs=pl.BlockSpec((BM, EMB), lambda i: (i, 0)),
    out_shape=jax.ShapeDtypeStruct((N_NODE, EMB), jnp.float32),
)


def _final_body(e_ref, x1_ref, pa_ref, pb_ref, s2_ref, out_ref):
    acc = e_ref[...] + x1_ref[...] + pa_ref[...] + pb_ref[...]
    out_ref[...] = lax.dot_general(acc, s2_ref[...], (((1,), (1,)), ((), ())),
                                   preferred_element_type=jnp.float32)


_final = pl.pallas_call(
    _final_body,
    grid=(N_NODE // BM,),
    in_specs=[pl.BlockSpec((BM, EMB), lambda i: (i, 0))] * 4
    + [pl.BlockSpec((N_SESS, EMB), lambda i: (0, 0))],
    out_specs=pl.BlockSpec((BM, N_SESS), lambda i: (i, 0)),
    out_shape=jax.ShapeDtypeStruct((N_NODE, N_SESS), jnp.float32),
)


def kernel(D, A, adj_rows, adj_cols, adj_vals, emb_table, emb1):
    pad = EPAD + 2 * CHUNK - N_EDGE
    rows_p = jnp.concatenate([adj_rows.astype(jnp.int32),
                              jnp.zeros((pad,), jnp.int32)])
    cols_p = jnp.concatenate([adj_cols.astype(jnp.int32),
                              jnp.zeros((pad,), jnp.int32)])
    vals_p = jnp.concatenate([adj_vals, jnp.zeros((pad,), jnp.float32)])

    p1a, p1b = _edge_pass(rows_p, cols_p, vals_p, emb_table)
    x1 = _combine(p1a, p1b)
    p2a, p2b = _edge_pass(rows_p, cols_p, vals_p, x1)
    sess = _line(D, A, emb1)
    return _final(emb_table, x1, p2a, p2b, sess)


# speedup vs baseline: 1.1149x; 1.1149x over previous
import functools

import jax
import jax.numpy as jnp
from jax import lax
from jax.experimental import pallas as pl
from jax.experimental.pallas import tpu as pltpu
from jax.experimental.pallas import tpu_sc as plsc

N_NODE = 10000
N_EDGE = 320000
EMB = 128
N_SESS = 1142

NC = 2
NS = 16
NW = NC * NS
CHUNK = 96
NCHK0 = 142
NCHK1 = 70
EPT0 = CHUNK * NCHK0
EPT1 = CHUNK * NCHK1
EPAD = NS * (EPT0 + EPT1)
WTILES = 10
RPT = N_NODE // WTILES
ZR = 200


def _edge_pass_body(rows_hbm, cols_hbm, vals_hbm, table_hbm, p0_hbm, p1_hbm,
                    idx_c0, idx_c1, idx_r0, idx_r1, val0, val1,
                    rowbuf0, rowbuf1, zbuf, acc,
                    semi0, semi1, semg0, semg1):
    c = lax.axis_index("c")
    s = lax.axis_index("s")
    nchk = jnp.where(c == 0, NCHK0, NCHK1)
    tbase = jnp.where(c == 0, s * EPT0, NS * EPT0 + s * EPT1)

    zv = jnp.zeros((16,), jnp.float32)

    def zrow(i, carry):
        for j in range(8):
            zbuf[i, pl.ds(j * 16, 16)] = zv
        return carry

    lax.fori_loop(0, ZR, zrow, 0)

    @pl.when(s < WTILES)
    def _():
        for q in range(RPT // ZR):
            off = pl.multiple_of(s * RPT + q * ZR, 8)
            pltpu.sync_copy(zbuf, acc.at[pl.ds(off, ZR)])

    plsc.subcore_barrier()

    idx_c = (idx_c0, idx_c1)
    idx_r = (idx_r0, idx_r1)
    val = (val0, val1)
    rowbuf = (rowbuf0, rowbuf1)
    semi = (semi0, semi1)
    semg = (semg0, semg1)

    def ebase(q):
        return pl.multiple_of(tbase + q * CHUNK, 8)

    def issue_idx(q, b):
        pltpu.async_copy(cols_hbm.at[pl.ds(ebase(q), CHUNK)], idx_c[b], semi[b])
        pltpu.async_copy(rows_hbm.at[pl.ds(ebase(q), CHUNK)], idx_r[b], semi[b])
        pltpu.async_copy(vals_hbm.at[pl.ds(ebase(q), CHUNK)], val[b], semi[b])

    def wait_idx(b):
        pltpu.make_async_copy(cols_hbm.at[pl.ds(0, CHUNK)], idx_c[b],
                              semi[b]).wait()
        pltpu.make_async_copy(rows_hbm.at[pl.ds(0, CHUNK)], idx_r[b],
                              semi[b]).wait()
        pltpu.make_async_copy(vals_hbm.at[pl.ds(0, CHUNK)], val[b],
                              semi[b]).wait()

    def issue_gather(b):
        pltpu.async_copy(table_hbm.at[idx_c[b]], rowbuf[b], semg[b])

    def wait_gather(b):
        pltpu.make_async_copy(table_hbm.at[pl.ds(0, CHUNK)], rowbuf[b],
                              semg[b]).wait()

    def process(b):
        @plsc.parallel_loop(0, CHUNK // 16, unroll=2)
        def _(g):
            vv = val[b][pl.ds(g * 16, 16)]
            for l in range(16):
                v = vv[l]
                e = g * 16 + l
                for j in range(8):
                    rowbuf[b][e, pl.ds(j * 16, 16)] = (
                        rowbuf[b][e, pl.ds(j * 16, 16)] * v)

        pltpu.sync_copy(rowbuf[b], acc.at[idx_r[b]], add=True)

    def slot_pair(k, carry):
        for b in range(2):
            t = k * 2 + b

            @pl.when(jnp.logical_and(t >= 1, t <= nchk))
            def _():
                wait_idx(1 - b)
                issue_gather(1 - b)

            @pl.when(t >= 2)
            def _():
                wait_gather(b)
                process(b)

            @pl.when(t <= nchk - 1)
            def _():
                issue_idx(t, b)
        return carry

    lax.fori_loop(0, nchk // 2 + 1, slot_pair, 0)
    plsc.subcore_barrier()

    woff = pl.multiple_of(s * RPT, 8)

    @pl.when(jnp.logical_and(s < WTILES, c == 0))
    def _():
        pltpu.sync_copy(acc.at[pl.ds(woff, RPT)], p0_hbm.at[pl.ds(woff, RPT)])

    @pl.when(jnp.logical_and(s < WTILES, c == 1))
    def _():
        pltpu.sync_copy(acc.at[pl.ds(woff, RPT)], p1_hbm.at[pl.ds(woff, RPT)])


_edge_pass = functools.partial(
    pl.kernel,
    out_type=(jax.ShapeDtypeStruct((N_NODE, EMB), jnp.float32),
              jax.ShapeDtypeStruct((N_NODE, EMB), jnp.float32)),
    mesh=plsc.VectorSubcoreMesh(core_axis_name="c", subcore_axis_name="s"),
    scratch_types=[
        pltpu.VMEM((CHUNK,), jnp.int32),
        pltpu.VMEM((CHUNK,), jnp.int32),
        pltpu.VMEM((CHUNK,), jnp.int32),
        pltpu.VMEM((CHUNK,), jnp.int32),
        pltpu.VMEM((CHUNK,), jnp.float32),
        pltpu.VMEM((CHUNK,), jnp.float32),
        pltpu.VMEM((CHUNK, EMB), jnp.float32),
        pltpu.VMEM((CHUNK, EMB), jnp.float32),
        pltpu.VMEM((ZR, EMB), jnp.float32),
        pltpu.VMEM_SHARED((N_NODE, EMB), jnp.float32),
        pltpu.SemaphoreType.DMA,
        pltpu.SemaphoreType.DMA,
        pltpu.SemaphoreType.DMA,
        pltpu.SemaphoreType.DMA,
    ],
)(_edge_pass_body)


def _line_body(d_ref, a_ref, e1_ref, out_ref):
    da = jnp.dot(d_ref[...], a_ref[...], preferred_element_type=jnp.float32)
    y1 = jnp.dot(da, e1_ref[...], preferred_element_type=jnp.float32)
    y2 = jnp.dot(da, y1, preferred_element_type=jnp.float32)
    out_ref[...] = e1_ref[...] + y1 + y2


_line = pl.pallas_call(
    _line_body,
    out_shape=jax.ShapeDtypeStruct((N_SESS, EMB), jnp.float32),
    in_specs=[pl.BlockSpec(memory_space=pltpu.VMEM)] * 3,
    out_specs=pl.BlockSpec(memory_space=pltpu.VMEM),
)

BM = 1000


def _add_body(a_ref, b_ref, o_ref):
    o_ref[...] = a_ref[...] + b_ref[...]


_combine = pl.pallas_call(
    _add_body,
    grid=(N_NODE // BM,),
    in_specs=[pl.BlockSpec((BM, EMB), lambda i: (i, 0))] * 2,
    out_specs=pl.BlockSpec((BM, EMB), lambda i: (i, 0)),
    out_shape=jax.ShapeDtypeStruct((N_NODE, EMB), jnp.float32),
)


def _final_body(e_ref, x1_ref, pa_ref, pb_ref, s2_ref, out_ref):
    acc = e_ref[...] + x1_ref[...] + pa_ref[...] + pb_ref[...]
    out_ref[...] = lax.dot_general(acc, s2_ref[...], (((1,), (1,)), ((), ())),
                                   preferred_element_type=jnp.float32)


_final = pl.pallas_call(
    _final_body,
    grid=(N_NODE // BM,),
    in_specs=[pl.BlockSpec((BM, EMB), lambda i: (i, 0))] * 4
    + [pl.BlockSpec((N_SESS, EMB), lambda i: (0, 0))],
    out_specs=pl.BlockSpec((BM, N_SESS), lambda i: (i, 0)),
    out_shape=jax.ShapeDtypeStruct((N_NODE, N_SESS), jnp.float32),
)


def kernel(D, A, adj_rows, adj_cols, adj_vals, emb_table, emb1):
    pad = EPAD + 2 * CHUNK - N_EDGE
    rows_p = jnp.concatenate([adj_rows.astype(jnp.int32),
                              jnp.zeros((pad,), jnp.int32)])
    cols_p = jnp.concatenate([adj_cols.astype(jnp.int32),
                              jnp.zeros((pad,), jnp.int32)])
    vals_p = jnp.concatenate([adj_vals, jnp.zeros((pad,), jnp.float32)])

    p1a, p1b = _edge_pass(rows_p, cols_p, vals_p, emb_table)
    x1 = _combine(p1a, p1b)
    p2a, p2b = _edge_pass(rows_p, cols_p, vals_p, x1)
    sess = _line(D, A, emb1)
    return _final(emb_table, x1, p2a, p2b, sess)


# speedup vs baseline: 1.1184x; 1.0032x over previous
import functools

import jax
import jax.numpy as jnp
from jax import lax
from jax.experimental import pallas as pl
from jax.experimental.pallas import tpu as pltpu
from jax.experimental.pallas import tpu_sc as plsc

N_NODE = 10000
N_EDGE = 320000
EMB = 128
N_SESS = 1142

NC = 2
NS = 16
NW = NC * NS
CHUNK = 96
NCHK0 = 166
NCHK1 = 46
EPT0 = CHUNK * NCHK0
EPT1 = CHUNK * NCHK1
EPAD = NS * (EPT0 + EPT1)
WTILES = 10
RPT = N_NODE // WTILES
ZR = 200


def _edge_pass_body(rows_hbm, cols_hbm, vals_hbm, table_hbm, p0_hbm, p1_hbm,
                    idx_c0, idx_c1, idx_r0, idx_r1, val0, val1,
                    rowbuf0, rowbuf1, zbuf, acc,
                    semi0, semi1, semg0, semg1):
    c = lax.axis_index("c")
    s = lax.axis_index("s")
    nchk = jnp.where(c == 0, NCHK0, NCHK1)
    tbase = jnp.where(c == 0, s * EPT0, NS * EPT0 + s * EPT1)

    zv = jnp.zeros((16,), jnp.float32)

    def zrow(i, carry):
        for j in range(8):
            zbuf[i, pl.ds(j * 16, 16)] = zv
        return carry

    lax.fori_loop(0, ZR, zrow, 0)

    @pl.when(s < WTILES)
    def _():
        for q in range(RPT // ZR):
            off = pl.multiple_of(s * RPT + q * ZR, 8)
            pltpu.sync_copy(zbuf, acc.at[pl.ds(off, ZR)])

    plsc.subcore_barrier()

    idx_c = (idx_c0, idx_c1)
    idx_r = (idx_r0, idx_r1)
    val = (val0, val1)
    rowbuf = (rowbuf0, rowbuf1)
    semi = (semi0, semi1)
    semg = (semg0, semg1)

    def ebase(q):
        return pl.multiple_of(tbase + q * CHUNK, 8)

    def issue_idx(q, b):
        pltpu.async_copy(cols_hbm.at[pl.ds(ebase(q), CHUNK)], idx_c[b], semi[b])
        pltpu.async_copy(rows_hbm.at[pl.ds(ebase(q), CHUNK)], idx_r[b], semi[b])
        pltpu.async_copy(vals_hbm.at[pl.ds(ebase(q), CHUNK)], val[b], semi[b])

    def wait_idx(b):
        pltpu.make_async_copy(cols_hbm.at[pl.ds(0, CHUNK)], idx_c[b],
                              semi[b]).wait()
        pltpu.make_async_copy(rows_hbm.at[pl.ds(0, CHUNK)], idx_r[b],
                              semi[b]).wait()
        pltpu.make_async_copy(vals_hbm.at[pl.ds(0, CHUNK)], val[b],
                              semi[b]).wait()

    def issue_gather(b):
        pltpu.async_copy(table_hbm.at[idx_c[b]], rowbuf[b], semg[b])

    def wait_gather(b):
        pltpu.make_async_copy(table_hbm.at[pl.ds(0, CHUNK)], rowbuf[b],
                              semg[b]).wait()

    def process(b):
        @plsc.parallel_loop(0, CHUNK // 16, unroll=2)
        def _(g):
            vv = val[b][pl.ds(g * 16, 16)]
            for l in range(16):
                v = vv[l]
                e = g * 16 + l
                for j in range(8):
                    rowbuf[b][e, pl.ds(j * 16, 16)] = (
                        rowbuf[b][e, pl.ds(j * 16, 16)] * v)

        pltpu.sync_copy(rowbuf[b], acc.at[idx_r[b]], add=True)

    def slot_pair(k, carry):
        for b in range(2):
            t = k * 2 + b

            @pl.when(jnp.logical_and(t >= 1, t <= nchk))
            def _():
                wait_idx(1 - b)
                issue_gather(1 - b)

            @pl.when(t >= 2)
            def _():
                wait_gather(b)
                process(b)

            @pl.when(t <= nchk - 1)
            def _():
                issue_idx(t, b)
        return carry

    lax.fori_loop(0, nchk // 2 + 1, slot_pair, 0)
    plsc.subcore_barrier()

    woff = pl.multiple_of(s * RPT, 8)

    @pl.when(jnp.logical_and(s < WTILES, c == 0))
    def _():
        pltpu.sync_copy(acc.at[pl.ds(woff, RPT)], p0_hbm.at[pl.ds(woff, RPT)])

    @pl.when(jnp.logical_and(s < WTILES, c == 1))
    def _():
        pltpu.sync_copy(acc.at[pl.ds(woff, RPT)], p1_hbm.at[pl.ds(woff, RPT)])


_edge_pass = functools.partial(
    pl.kernel,
    out_type=(jax.ShapeDtypeStruct((N_NODE, EMB), jnp.float32),
              jax.ShapeDtypeStruct((N_NODE, EMB), jnp.float32)),
    mesh=plsc.VectorSubcoreMesh(core_axis_name="c", subcore_axis_name="s"),
    scratch_types=[
        pltpu.VMEM((CHUNK,), jnp.int32),
        pltpu.VMEM((CHUNK,), jnp.int32),
        pltpu.VMEM((CHUNK,), jnp.int32),
        pltpu.VMEM((CHUNK,), jnp.int32),
        pltpu.VMEM((CHUNK,), jnp.float32),
        pltpu.VMEM((CHUNK,), jnp.float32),
        pltpu.VMEM((CHUNK, EMB), jnp.float32),
        pltpu.VMEM((CHUNK, EMB), jnp.float32),
        pltpu.VMEM((ZR, EMB), jnp.float32),
        pltpu.VMEM_SHARED((N_NODE, EMB), jnp.float32),
        pltpu.SemaphoreType.DMA,
        pltpu.SemaphoreType.DMA,
        pltpu.SemaphoreType.DMA,
        pltpu.SemaphoreType.DMA,
    ],
)(_edge_pass_body)


def _line_body(d_ref, a_ref, e1_ref, out_ref):
    da = jnp.dot(d_ref[...], a_ref[...], preferred_element_type=jnp.float32)
    y1 = jnp.dot(da, e1_ref[...], preferred_element_type=jnp.float32)
    y2 = jnp.dot(da, y1, preferred_element_type=jnp.float32)
    out_ref[...] = e1_ref[...] + y1 + y2


_line = pl.pallas_call(
    _line_body,
    out_shape=jax.ShapeDtypeStruct((N_SESS, EMB), jnp.float32),
    in_specs=[pl.BlockSpec(memory_space=pltpu.VMEM)] * 3,
    out_specs=pl.BlockSpec(memory_space=pltpu.VMEM),
)

BM = 1000


def _add_body(a_ref, b_ref, o_ref):
    o_ref[...] = a_ref[...] + b_ref[...]


_combine = pl.pallas_call(
    _add_body,
    grid=(N_NODE // BM,),
    in_specs=[pl.BlockSpec((BM, EMB), lambda i: (i, 0))] * 2,
    out_specs=pl.BlockSpec((BM, EMB), lambda i: (i, 0)),
    out_shape=jax.ShapeDtypeStruct((N_NODE, EMB), jnp.float32),
)


def _final_body(e_ref, x1_ref, pa_ref, pb_ref, s2_ref, out_ref):
    acc = e_ref[...] + x1_ref[...] + pa_ref[...] + pb_ref[...]
    out_ref[...] = lax.dot_general(acc, s2_ref[...], (((1,), (1,)), ((), ())),
                                   preferred_element_type=jnp.float32)


_final = pl.pallas_call(
    _final_body,
    grid=(N_NODE // BM,),
    in_specs=[pl.BlockSpec((BM, EMB), lambda i: (i, 0))] * 4
    + [pl.BlockSpec((N_SESS, EMB), lambda i: (0, 0))],
    out_specs=pl.BlockSpec((BM, N_SESS), lambda i: (i, 0)),
    out_shape=jax.ShapeDtypeStruct((N_NODE, N_SESS), jnp.float32),
)


def kernel(D, A, adj_rows, adj_cols, adj_vals, emb_table, emb1):
    pad = EPAD + 2 * CHUNK - N_EDGE
    rows_p = jnp.concatenate([adj_rows.astype(jnp.int32),
                              jnp.zeros((pad,), jnp.int32)])
    cols_p = jnp.concatenate([adj_cols.astype(jnp.int32),
                              jnp.zeros((pad,), jnp.int32)])
    vals_p = jnp.concatenate([adj_vals, jnp.zeros((pad,), jnp.float32)])

    p1a, p1b = _edge_pass(rows_p, cols_p, vals_p, emb_table)
    x1 = _combine(p1a, p1b)
    p2a, p2b = _edge_pass(rows_p, cols_p, vals_p, x1)
    sess = _line(D, A, emb1)
    return _final(emb_table, x1, p2a, p2b, sess)
